# baseline (speedup 1.0000x reference)
import functools

import jax
import jax.numpy as jnp
from jax import lax
from jax.experimental import pallas as pl
from jax.experimental.pallas import tpu as pltpu
from jax.experimental.pallas import tpu_sc as plsc

B, C, D = 4096, 64, 64
CSC = 32
NW = 32
BW = B // NW
NBUF = 4
GRPS = BW // 16
REJECTION_MARGIN = 0.3


def _bce_body(logits_ref, labels_ref, out_ref):
    x = logits_ref[...]
    y = labels_ref[...]
    bce = jnp.maximum(x, 0.0) - x * y + jnp.log1p(jnp.exp(-jnp.abs(x)))
    out_ref[...] = jnp.sum(bce, axis=0)


def _bce_per_sample(logits_t, labels_t):
    return pl.pallas_call(
        _bce_body,
        out_shape=jax.ShapeDtypeStruct((B,), jnp.float32),
    )(logits_t, labels_t)


def _tc_body(bce_ref, labels_ref, wf_ref, out_ref):
    i = pl.program_id(0)

    @pl.when(i == 0)
    def _():
        out_ref[...] = bce_ref[...]

    t = jnp.maximum(wf_ref[0, 0:8], wf_ref[0, 8:16])
    for j in range(2, D // 8):
        t = jnp.maximum(t, wf_ref[0, 8 * j:8 * j + 8])
    sim = jnp.max(t, axis=0)
    rej = jnp.maximum(jax.nn.sigmoid(sim) - REJECTION_MARGIN, 0.0)
    mask = 1.0 - labels_ref[pl.ds(CSC + i, 1), :][0]
    out_ref[...] += rej * mask


def _tc_partial(bce, labels_t, wf_t):
    return pl.pallas_call(
        _tc_body,
        grid=(C - CSC,),
        in_specs=[
            pl.BlockSpec((B,), lambda i: (0,)),
            pl.BlockSpec((C, B), lambda i: (0, 0)),
            pl.BlockSpec((1, D, B), lambda i: (CSC + i, 0, 0)),
        ],
        out_specs=pl.BlockSpec((B,), lambda i: (0,)),
        out_shape=jax.ShapeDtypeStruct((B,), jnp.float32),
    )(bce, labels_t, wf_t)


def _add_body(a_ref, b_ref, out_ref):
    out_ref[...] = a_ref[...] + b_ref[...]


def _tc_add(a, b):
    return pl.pallas_call(
        _add_body,
        out_shape=jax.ShapeDtypeStruct((B,), jnp.float32),
    )(a, b)


def _sc_body(wf_hbm, labels_hbm, out_hbm, wbuf, lab_v, acc_v, sems):
    cid = lax.axis_index("c")
    sid = lax.axis_index("s")
    wid = sid * 2 + cid
    b0 = wid * BW

    pltpu.sync_copy(labels_hbm.at[:, pl.ds(b0, BW)], lab_v)
    for g in range(GRPS):
        acc_v[pl.ds(g * 16, 16)] = jnp.zeros((16,), jnp.float32)

    def wf_dma(c2, k):
        return pltpu.make_async_copy(
            wf_hbm.at[pl.ds(c2 * 2, 2), :, pl.ds(b0, BW)],
            wbuf.at[k], sems.at[k])

    for k in range(NBUF):
        wf_dma(k, k).start()

    def compute_slab(c2, k):
        for half in range(2):
            c = c2 * 2 + half
            buf = wbuf.at[k, half]

            def grp_body(g, _):
                s = pl.ds(g * 16, 16)
                accs = [buf[d, s] for d in range(4)]
                for d in range(4, D):
                    accs[d % 4] = jnp.maximum(accs[d % 4], buf[d, s])
                m = jnp.maximum(jnp.maximum(accs[0], accs[1]),
                                jnp.maximum(accs[2], accs[3]))
                sig = 1.0 / (1.0 + jnp.exp(-m))
                rej = jnp.maximum(sig - REJECTION_MARGIN, 0.0)
                contrib = rej * (1.0 - lab_v[c, s])
                plsc.addupdate(acc_v.at[s], contrib)
                return 0

            lax.fori_loop(0, GRPS, grp_body, 0)

    NC2 = CSC // 2

    def outer(gidx, _):
        for k in range(NBUF):
            c2 = gidx * NBUF + k
            wf_dma(c2, k).wait()
            compute_slab(c2, k)
            nc2 = c2 + NBUF

            @pl.when(nc2 < NC2)
            def _():
                wf_dma(nc2, k).start()
        return 0

    lax.fori_loop(0, NC2 // NBUF, outer, 0)

    pltpu.sync_copy(acc_v, out_hbm.at[pl.ds(b0, BW)])


@functools.partial(
    pl.kernel,
    mesh=plsc.VectorSubcoreMesh(core_axis_name="c", subcore_axis_name="s"),
    out_type=jax.ShapeDtypeStruct((B,), jnp.float32),
    scratch_types=[
        pltpu.VMEM((NBUF, 2, D, BW), jnp.float32),
        pltpu.VMEM((C, BW), jnp.float32),
        pltpu.VMEM((BW,), jnp.float32),
        pltpu.SemaphoreType.DMA((NBUF,)),
    ],
    compiler_params=pltpu.CompilerParams(needs_layout_passes=False),
)
def _sc_rejection(wf_hbm, labels_hbm, out_hbm, wbuf, lab_v, acc_v, sems):
    _sc_body(wf_hbm, labels_hbm, out_hbm, wbuf, lab_v, acc_v, sems)


def kernel(logits, wf, labels):
    logits_t = jnp.transpose(logits)
    labels_t = jnp.transpose(labels)
    wf_t = jnp.transpose(wf, (0, 2, 1))
    rej_sc = _sc_rejection(wf_t, labels_t)
    bce = _bce_per_sample(logits_t, labels_t)
    partial = _tc_partial(bce, labels_t, wf_t)
    return _tc_add(partial, rej_sc)


# speedup vs baseline: 1.0573x; 1.0573x over previous
import functools

import jax
import jax.numpy as jnp
from jax import lax
from jax.experimental import pallas as pl
from jax.experimental.pallas import tpu as pltpu
from jax.experimental.pallas import tpu_sc as plsc

B, C, D = 4096, 64, 64
CSC = 40
NW = 32
BW = B // NW
NBUF = 4
GRPS = BW // 16
REJECTION_MARGIN = 0.3


def _bce_body(logits_ref, labels_ref, out_ref):
    x = logits_ref[...]
    y = labels_ref[...]
    bce = jnp.maximum(x, 0.0) - x * y + jnp.log1p(jnp.exp(-jnp.abs(x)))
    out_ref[...] = jnp.sum(bce, axis=0)


def _bce_per_sample(logits_t, labels_t):
    return pl.pallas_call(
        _bce_body,
        out_shape=jax.ShapeDtypeStruct((B,), jnp.float32),
    )(logits_t, labels_t)


def _tc_body(bce_ref, labels_ref, wf_ref, out_ref):
    i = pl.program_id(0)

    @pl.when(i == 0)
    def _():
        out_ref[...] = bce_ref[...]

    t = jnp.maximum(wf_ref[0, 0:8], wf_ref[0, 8:16])
    for j in range(2, D // 8):
        t = jnp.maximum(t, wf_ref[0, 8 * j:8 * j + 8])
    sim = jnp.max(t, axis=0)
    rej = jnp.maximum(jax.nn.sigmoid(sim) - REJECTION_MARGIN, 0.0)
    mask = 1.0 - labels_ref[pl.ds(CSC + i, 1), :][0]
    out_ref[...] += rej * mask


def _tc_partial(bce, labels_t, wf_t):
    return pl.pallas_call(
        _tc_body,
        grid=(C - CSC,),
        in_specs=[
            pl.BlockSpec((B,), lambda i: (0,)),
            pl.BlockSpec((C, B), lambda i: (0, 0)),
            pl.BlockSpec((1, D, B), lambda i: (CSC + i, 0, 0)),
        ],
        out_specs=pl.BlockSpec((B,), lambda i: (0,)),
        out_shape=jax.ShapeDtypeStruct((B,), jnp.float32),
    )(bce, labels_t, wf_t)


def _add_body(a_ref, b_ref, out_ref):
    out_ref[...] = a_ref[...] + b_ref[...]


def _tc_add(a, b):
    return pl.pallas_call(
        _add_body,
        out_shape=jax.ShapeDtypeStruct((B,), jnp.float32),
    )(a, b)


def _sc_body(wf_hbm, labels_hbm, out_hbm, wbuf, lab_v, acc_v, sems):
    cid = lax.axis_index("c")
    sid = lax.axis_index("s")
    wid = sid * 2 + cid
    b0 = wid * BW

    pltpu.sync_copy(labels_hbm.at[:, pl.ds(b0, BW)], lab_v)
    for g in range(GRPS):
        acc_v[pl.ds(g * 16, 16)] = jnp.zeros((16,), jnp.float32)

    def wf_dma(c2, k):
        return pltpu.make_async_copy(
            wf_hbm.at[pl.ds(c2 * 2, 2), :, pl.ds(b0, BW)],
            wbuf.at[k], sems.at[k])

    for k in range(NBUF):
        wf_dma(k, k).start()

    def compute_slab(c2, k):
        for half in range(2):
            c = c2 * 2 + half
            buf = wbuf.at[k, half]

            def grp_body(g, _):
                s = pl.ds(g * 16, 16)
                accs = [buf[d, s] for d in range(4)]
                for d in range(4, D):
                    accs[d % 4] = jnp.maximum(accs[d % 4], buf[d, s])
                m = jnp.maximum(jnp.maximum(accs[0], accs[1]),
                                jnp.maximum(accs[2], accs[3]))
                sig = 1.0 / (1.0 + jnp.exp(-m))
                rej = jnp.maximum(sig - REJECTION_MARGIN, 0.0)
                contrib = rej * (1.0 - lab_v[c, s])
                plsc.addupdate(acc_v.at[s], contrib)
                return 0

            lax.fori_loop(0, GRPS, grp_body, 0)

    NC2 = CSC // 2

    def outer(gidx, _):
        for k in range(NBUF):
            c2 = gidx * NBUF + k
            wf_dma(c2, k).wait()
            compute_slab(c2, k)
            nc2 = c2 + NBUF

            @pl.when(nc2 < NC2)
            def _():
                wf_dma(nc2, k).start()
        return 0

    lax.fori_loop(0, NC2 // NBUF, outer, 0)

    pltpu.sync_copy(acc_v, out_hbm.at[pl.ds(b0, BW)])


@functools.partial(
    pl.kernel,
    mesh=plsc.VectorSubcoreMesh(core_axis_name="c", subcore_axis_name="s"),
    out_type=jax.ShapeDtypeStruct((B,), jnp.float32),
    scratch_types=[
        pltpu.VMEM((NBUF, 2, D, BW), jnp.float32),
        pltpu.VMEM((C, BW), jnp.float32),
        pltpu.VMEM((BW,), jnp.float32),
        pltpu.SemaphoreType.DMA((NBUF,)),
    ],
    compiler_params=pltpu.CompilerParams(needs_layout_passes=False),
)
def _sc_rejection(wf_hbm, labels_hbm, out_hbm, wbuf, lab_v, acc_v, sems):
    _sc_body(wf_hbm, labels_hbm, out_hbm, wbuf, lab_v, acc_v, sems)


def kernel(logits, wf, labels):
    logits_t = jnp.transpose(logits)
    labels_t = jnp.transpose(labels)
    wf_t = jnp.transpose(wf, (0, 2, 1))
    rej_sc = _sc_rejection(wf_t, labels_t)
    bce = _bce_per_sample(logits_t, labels_t)
    partial = _tc_partial(bce, labels_t, wf_t)
    return _tc_add(partial, rej_sc)
